# baseline (speedup 1.0000x reference)
import functools

import jax
import jax.numpy as jnp
from jax import lax
from jax.experimental import pallas as pl
from jax.experimental.pallas import tpu as pltpu
from jax.experimental.pallas import tpu_sc as plsc

NC = 2
NS = 16
NW = NC * NS


def _sc_gather(user_table, item_table, user, item, chunk_base, chunk_rows):
    D = user_table.shape[1]
    b_per_w = chunk_rows // NW
    mesh = plsc.VectorSubcoreMesh(core_axis_name="c", subcore_axis_name="s")

    @functools.partial(
        pl.kernel,
        mesh=mesh,
        out_type=(
            jax.ShapeDtypeStruct((chunk_rows, D), jnp.float32),
            jax.ShapeDtypeStruct((chunk_rows, D), jnp.float32),
        ),
        scratch_types=[
            pltpu.VMEM((b_per_w,), jnp.int32),
            pltpu.VMEM((b_per_w,), jnp.int32),
            pltpu.VMEM((b_per_w, D), jnp.float32),
            pltpu.VMEM((b_per_w, D), jnp.float32),
            pltpu.SemaphoreType.DMA((4,)),
            pltpu.SemaphoreType.DMA((4,)),
        ],
    )
    def gather_kernel(ut_hbm, it_hbm, uidx_hbm, iidx_hbm, uo_hbm, io_hbm,
                      uidx_v, iidx_v, urows_v, irows_v, usems, isems):
        wid = lax.axis_index("s") * NC + lax.axis_index("c")
        base = wid * b_per_w
        nsub = 4
        sub = b_per_w // nsub
        lu = pltpu.async_copy(uidx_hbm.at[pl.ds(chunk_base + base, b_per_w)],
                              uidx_v, usems.at[0])
        li = pltpu.async_copy(iidx_hbm.at[pl.ds(chunk_base + base, b_per_w)],
                              iidx_v, isems.at[0])
        lu.wait()
        li.wait()
        gathers = []
        for s in range(nsub):
            sl = pl.ds(s * sub, sub)
            gathers.append(
                (sl,
                 pltpu.async_copy(ut_hbm.at[uidx_v.at[sl]], urows_v.at[sl],
                                  usems.at[s]),
                 pltpu.async_copy(it_hbm.at[iidx_v.at[sl]], irows_v.at[sl],
                                  isems.at[s])))
        writes = []
        for s, (sl, ug, ig) in enumerate(gathers):
            dst = pl.ds(base + s * sub, sub)
            ug.wait()
            writes.append(pltpu.async_copy(urows_v.at[sl], uo_hbm.at[dst],
                                           usems.at[s]))
            ig.wait()
            writes.append(pltpu.async_copy(irows_v.at[sl], io_hbm.at[dst],
                                           isems.at[s]))
        for w in writes:
            w.wait()

    return gather_kernel(user_table, item_table, user, item)


def _mlp_body(ue_ref, ie_ref, w1a_ref, b1_ref, w2_ref, b2_ref,
              wfa_ref, wfb_ref, bf_ref, *rest):
    out_ref = rest[-1]
    ueb = ue_ref[...].astype(jnp.bfloat16)
    ieb = ie_ref[...].astype(jnp.bfloat16)
    x = jnp.concatenate([ueb, ieb], axis=1)
    h1 = (jnp.dot(x, w1a_ref[...], preferred_element_type=jnp.float32)
          + b1_ref[...]).astype(jnp.bfloat16)
    h1 = jnp.maximum(h1, jnp.bfloat16(0.0))
    h2 = (jnp.dot(h1, w2_ref[...], preferred_element_type=jnp.float32)
          + b2_ref[...]).astype(jnp.bfloat16)
    h2 = jnp.maximum(h2, jnp.bfloat16(0.0))
    gmf = ueb * ieb
    out = (jnp.dot(gmf, wfa_ref[...], preferred_element_type=jnp.float32)
           + jnp.dot(h2, wfb_ref[...], preferred_element_type=jnp.float32))
    out_ref[...] = (out + bf_ref[0]).T[None]


def _prep_weights(D, W1, b1, W2, b2, Wf):
    H = W1.shape[0]
    w1 = W1.T.astype(jnp.bfloat16)
    w2 = W2.T.astype(jnp.bfloat16)
    wfa = Wf[:, :D].T.astype(jnp.bfloat16)
    wfb = Wf[:, D:].T.astype(jnp.bfloat16)
    b1r = b1.reshape(1, H)
    b2r = b2.reshape(1, W2.shape[0])
    return w1, b1r, w2, b2r, wfa, wfb


def _tc_mlp(ue, ie, w1, b1r, w2, b2r, wfa, wfb, bf, nb_total, block_off,
            carry=None, block_b=2048):
    Bc, D = ue.shape
    H = w1.shape[1]
    D2 = w2.shape[1]
    nb = Bc // block_b

    in_specs = [
        pl.BlockSpec((block_b, D), lambda i: (i, 0)),
        pl.BlockSpec((block_b, D), lambda i: (i, 0)),
        pl.BlockSpec((2 * D, H), lambda i: (0, 0)),
        pl.BlockSpec((1, H), lambda i: (0, 0)),
        pl.BlockSpec((H, D2), lambda i: (0, 0)),
        pl.BlockSpec((1, D2), lambda i: (0, 0)),
        pl.BlockSpec((D, 1), lambda i: (0, 0)),
        pl.BlockSpec((D2, 1), lambda i: (0, 0)),
        pl.BlockSpec((1,), lambda i: (0,)),
    ]
    args = [ue, ie, w1, b1r, w2, b2r, wfa, wfb, bf]
    aliases = {}
    if carry is not None:
        in_specs.append(pl.BlockSpec(memory_space=pl.ANY))
        args.append(carry)
        aliases = {9: 0}
    return pl.pallas_call(
        _mlp_body,
        grid=(nb,),
        in_specs=in_specs,
        out_specs=pl.BlockSpec((1, 1, block_b),
                               lambda i: (i + block_off, 0, 0)),
        out_shape=jax.ShapeDtypeStruct((nb_total, 1, block_b), jnp.float32),
        input_output_aliases=aliases,
    )(*args)


NCHUNKS = 2
BLOCK_B = 2048


@jax.jit
def kernel(user, item, user_table, item_table, W1, b1, W2, b2, Wf, bf):
    B = user.shape[0]
    D = user_table.shape[1]
    wp = _prep_weights(D, W1, b1, W2, b2, Wf)
    Bc = B // NCHUNKS
    nb_total = B // BLOCK_B
    nb_chunk = Bc // BLOCK_B
    embs = [
        _sc_gather(user_table, item_table, user, item, c * Bc, Bc)
        for c in range(NCHUNKS)
    ]
    out = None
    for c, (ue, ie) in enumerate(embs):
        out = _tc_mlp(ue, ie, *wp, bf, nb_total, c * nb_chunk, carry=out,
                      block_b=BLOCK_B)
    return out.reshape(B)


# speedup vs baseline: 1.0109x; 1.0109x over previous
import functools

import jax
import jax.numpy as jnp
from jax import lax
from jax.experimental import pallas as pl
from jax.experimental.pallas import tpu as pltpu
from jax.experimental.pallas import tpu_sc as plsc

NC = 2
NS = 16
NW = NC * NS


def _sc_gather(user_table, item_table, user, item, chunk_base, chunk_rows):
    D = user_table.shape[1]
    b_per_w = chunk_rows // NW
    mesh = plsc.VectorSubcoreMesh(core_axis_name="c", subcore_axis_name="s")

    @functools.partial(
        pl.kernel,
        mesh=mesh,
        out_type=(
            jax.ShapeDtypeStruct((chunk_rows, D), jnp.float32),
            jax.ShapeDtypeStruct((chunk_rows, D), jnp.float32),
        ),
        scratch_types=[
            pltpu.VMEM((b_per_w,), jnp.int32),
            pltpu.VMEM((b_per_w,), jnp.int32),
            pltpu.VMEM((b_per_w, D), jnp.float32),
            pltpu.VMEM((b_per_w, D), jnp.float32),
            pltpu.SemaphoreType.DMA((2,)),
            pltpu.SemaphoreType.DMA((2,)),
        ],
    )
    def gather_kernel(ut_hbm, it_hbm, uidx_hbm, iidx_hbm, uo_hbm, io_hbm,
                      uidx_v, iidx_v, urows_v, irows_v, usems, isems):
        wid = lax.axis_index("s") * NC + lax.axis_index("c")
        base = wid * b_per_w
        nsub = 2
        sub = b_per_w // nsub
        lu = pltpu.async_copy(uidx_hbm.at[pl.ds(chunk_base + base, b_per_w)],
                              uidx_v, usems.at[0])
        li = pltpu.async_copy(iidx_hbm.at[pl.ds(chunk_base + base, b_per_w)],
                              iidx_v, isems.at[0])
        lu.wait()
        li.wait()
        gathers = []
        for s in range(nsub):
            sl = pl.ds(s * sub, sub)
            gathers.append(
                (sl,
                 pltpu.async_copy(ut_hbm.at[uidx_v.at[sl]], urows_v.at[sl],
                                  usems.at[s]),
                 pltpu.async_copy(it_hbm.at[iidx_v.at[sl]], irows_v.at[sl],
                                  isems.at[s])))
        writes = []
        for s, (sl, ug, ig) in enumerate(gathers):
            dst = pl.ds(base + s * sub, sub)
            ug.wait()
            writes.append(pltpu.async_copy(urows_v.at[sl], uo_hbm.at[dst],
                                           usems.at[s]))
            ig.wait()
            writes.append(pltpu.async_copy(irows_v.at[sl], io_hbm.at[dst],
                                           isems.at[s]))
        for w in writes:
            w.wait()

    return gather_kernel(user_table, item_table, user, item)


def _mlp_body(ue_ref, ie_ref, w1a_ref, b1_ref, w2_ref, b2_ref,
              wfa_ref, wfb_ref, bf_ref, *rest):
    out_ref = rest[-1]
    ueb = ue_ref[...].astype(jnp.bfloat16)
    ieb = ie_ref[...].astype(jnp.bfloat16)
    x = jnp.concatenate([ueb, ieb], axis=1)
    h1 = (jnp.dot(x, w1a_ref[...], preferred_element_type=jnp.float32)
          + b1_ref[...]).astype(jnp.bfloat16)
    h1 = jnp.maximum(h1, jnp.bfloat16(0.0))
    h2 = (jnp.dot(h1, w2_ref[...], preferred_element_type=jnp.float32)
          + b2_ref[...]).astype(jnp.bfloat16)
    h2 = jnp.maximum(h2, jnp.bfloat16(0.0))
    gmf = ueb * ieb
    out = (jnp.dot(gmf, wfa_ref[...], preferred_element_type=jnp.float32)
           + jnp.dot(h2, wfb_ref[...], preferred_element_type=jnp.float32))
    out_ref[...] = (out + bf_ref[0]).T[None]


def _prep_weights(D, W1, b1, W2, b2, Wf):
    H = W1.shape[0]
    w1 = W1.T.astype(jnp.bfloat16)
    w2 = W2.T.astype(jnp.bfloat16)
    wfa = Wf[:, :D].T.astype(jnp.bfloat16)
    wfb = Wf[:, D:].T.astype(jnp.bfloat16)
    b1r = b1.reshape(1, H)
    b2r = b2.reshape(1, W2.shape[0])
    return w1, b1r, w2, b2r, wfa, wfb


def _tc_mlp(ue, ie, w1, b1r, w2, b2r, wfa, wfb, bf, nb_total, block_off,
            carry=None, block_b=2048):
    Bc, D = ue.shape
    H = w1.shape[1]
    D2 = w2.shape[1]
    nb = Bc // block_b

    in_specs = [
        pl.BlockSpec((block_b, D), lambda i: (i, 0)),
        pl.BlockSpec((block_b, D), lambda i: (i, 0)),
        pl.BlockSpec((2 * D, H), lambda i: (0, 0)),
        pl.BlockSpec((1, H), lambda i: (0, 0)),
        pl.BlockSpec((H, D2), lambda i: (0, 0)),
        pl.BlockSpec((1, D2), lambda i: (0, 0)),
        pl.BlockSpec((D, 1), lambda i: (0, 0)),
        pl.BlockSpec((D2, 1), lambda i: (0, 0)),
        pl.BlockSpec((1,), lambda i: (0,)),
    ]
    args = [ue, ie, w1, b1r, w2, b2r, wfa, wfb, bf]
    aliases = {}
    if carry is not None:
        in_specs.append(pl.BlockSpec(memory_space=pl.ANY))
        args.append(carry)
        aliases = {9: 0}
    return pl.pallas_call(
        _mlp_body,
        grid=(nb,),
        in_specs=in_specs,
        out_specs=pl.BlockSpec((1, 1, block_b),
                               lambda i: (i + block_off, 0, 0)),
        out_shape=jax.ShapeDtypeStruct((nb_total, 1, block_b), jnp.float32),
        input_output_aliases=aliases,
    )(*args)


NCHUNKS = 2
BLOCK_B = 2048


@jax.jit
def kernel(user, item, user_table, item_table, W1, b1, W2, b2, Wf, bf):
    B = user.shape[0]
    D = user_table.shape[1]
    wp = _prep_weights(D, W1, b1, W2, b2, Wf)
    Bc = B // NCHUNKS
    nb_total = B // BLOCK_B
    nb_chunk = Bc // BLOCK_B
    embs = [
        _sc_gather(user_table, item_table, user, item, c * Bc, Bc)
        for c in range(NCHUNKS)
    ]
    out = None
    for c, (ue, ie) in enumerate(embs):
        out = _tc_mlp(ue, ie, *wp, bf, nb_total, c * nb_chunk, carry=out,
                      block_b=BLOCK_B)
    return out.reshape(B)
